# Initial kernel scaffold; baseline (speedup 1.0000x reference)
import jax
import jax.numpy as jnp
from jax.experimental import pallas as pl


def kernel(input, weight):
    raise NotImplementedError("write your pallas kernel here")


# speedup vs baseline: 1.8742x; 1.8742x over previous
import functools

import jax
import jax.numpy as jnp
from jax import lax
from jax.experimental import pallas as pl
from jax.experimental.pallas import tpu as pltpu
from jax.experimental.pallas import tpu_sc as plsc

NINP = 64
NW = 32
CHUNK = 128
NBUF = 8


@functools.partial(jax.jit, static_argnames=("n_chunks",))
def _sc_gather(weight, idx3, *, n_chunks):
    mesh = plsc.VectorSubcoreMesh(core_axis_name="c", subcore_axis_name="s")
    n_groups = n_chunks // NBUF

    def body(weight_hbm, idx_hbm, out_hbm, idx_v, bufs, *gsems):
        wid = lax.axis_index("s") * 2 + lax.axis_index("c")
        pltpu.sync_copy(idx_hbm.at[wid], idx_v)

        def start_gather(j, b):
            pltpu.async_copy(weight_hbm.at[idx_v.at[j]], bufs.at[b], gsems[b])

        def wait_gather(b):
            pltpu.make_async_copy(
                weight_hbm.at[pl.ds(0, CHUNK)], bufs.at[b], gsems[b]
            ).wait()

        for b in range(NBUF):
            start_gather(b, b)

        def group(g, carry):
            for b in range(NBUF):
                j = g * NBUF + b
                wait_gather(b)
                pltpu.sync_copy(bufs.at[b], out_hbm.at[wid, j])
                nj = j + NBUF

                @pl.when(nj < n_chunks)
                def _():
                    start_gather(nj, b)

            return carry

        lax.fori_loop(0, n_groups, group, 0)
        if n_chunks % NBUF:
            for b in range(n_chunks % NBUF):
                j = n_groups * NBUF + b
                wait_gather(b)
                pltpu.sync_copy(bufs.at[b], out_hbm.at[wid, j])

    call = pl.kernel(
        body,
        out_type=jax.ShapeDtypeStruct((NW, n_chunks, CHUNK, NINP), jnp.float32),
        mesh=mesh,
        compiler_params=pltpu.CompilerParams(use_tc_tiling_on_sc=False),
        scratch_types=[
            pltpu.VMEM((n_chunks, CHUNK), jnp.int32),
            pltpu.VMEM((NBUF, CHUNK, NINP), jnp.float32),
        ] + [pltpu.SemaphoreType.DMA] * NBUF,
    )
    return call(weight, idx3)


def kernel(input, weight):
    b, h = input.shape
    n = b * h
    idxf = input.reshape(n).astype(jnp.int32)
    rem = (-n) % (NW * CHUNK)
    if rem:
        idxf = jnp.pad(idxf, (0, rem))
    n_chunks = (n + rem) // (NW * CHUNK)
    idx3 = idxf.reshape(NW, n_chunks, CHUNK)
    out = _sc_gather(weight, idx3, n_chunks=n_chunks)
    return out.reshape(NW * n_chunks * CHUNK, NINP)[:n].reshape(b, h, NINP)
